# Initial kernel scaffold; baseline (speedup 1.0000x reference)
import jax
import jax.numpy as jnp
from jax.experimental import pallas as pl


def kernel(q, P):
    raise NotImplementedError("write your pallas kernel here")


# speedup vs baseline: 1.1711x; 1.1711x over previous
import functools

import jax
import jax.numpy as jnp
from jax.experimental import pallas as pl

NP_ = 32
NK_ = 32
M_ = NP_ * NK_
EMBED_ = 256
BR_ = 1024


def _body(q_ref, P_ref, pred_ref, D_ref, E_ref):
    q = q_ref[...]
    Pm = P_ref[...]
    S = jax.lax.dot_general(
        q, Pm, (((1,), (1,)), ((), ())),
        preferred_element_type=jnp.float32)
    q2 = jnp.sum(q * q, axis=1, keepdims=True)
    p2 = jnp.sum(Pm * Pm, axis=1)[None, :]
    D2 = jnp.maximum(q2 + p2 - 2.0 * S, 0.0)
    D = jnp.sqrt(D2 + 1e-12)
    E = jnp.exp(-D)
    D_ref[...] = D
    E_ref[...] = E
    col = jax.lax.broadcasted_iota(jnp.int32, (M_, NP_), 0)
    grp = jax.lax.broadcasted_iota(jnp.int32, (M_, NP_), 1)
    G = jnp.where(col // NK_ == grp, 1.0, 0.0).astype(jnp.float32)
    ps = jax.lax.dot_general(
        E, G, (((1,), (0,)), ((), ())),
        preferred_element_type=jnp.float32)
    pred_ref[...] = ps / jnp.sum(ps, axis=1, keepdims=True)


@functools.partial(jax.jit, static_argnames=())
def kernel(q, P):
    B = q.shape[0]
    nb = B // BR_
    pred, D, E = pl.pallas_call(
        _body,
        grid=(nb,),
        in_specs=[
            pl.BlockSpec((BR_, EMBED_), lambda i: (i, 0)),
            pl.BlockSpec((M_, EMBED_), lambda i: (0, 0)),
        ],
        out_specs=[
            pl.BlockSpec((BR_, NP_), lambda i: (i, 0)),
            pl.BlockSpec((BR_, M_), lambda i: (i, 0)),
            pl.BlockSpec((BR_, M_), lambda i: (i, 0)),
        ],
        out_shape=[
            jax.ShapeDtypeStruct((B, NP_), jnp.float32),
            jax.ShapeDtypeStruct((B, M_), jnp.float32),
            jax.ShapeDtypeStruct((B, M_), jnp.float32),
        ],
    )(q, P)
    return (pred, D, E.reshape(B, NP_, NK_))


# speedup vs baseline: 1.2250x; 1.0460x over previous
import functools

import jax
import jax.numpy as jnp
from jax.experimental import pallas as pl
from jax.experimental.pallas import tpu as pltpu

NP_ = 32
NK_ = 32
M_ = NP_ * NK_
EMBED_ = 256
BR_ = 1024


def _body(q_ref, P_ref, pred_ref, D_ref, E_ref):
    q = q_ref[...]
    Pm = P_ref[...]
    qb = (q * -2.0).astype(jnp.bfloat16)
    Pb = Pm.astype(jnp.bfloat16)
    S = jax.lax.dot_general(
        qb, Pb, (((1,), (1,)), ((), ())),
        preferred_element_type=jnp.float32)
    q2 = jnp.sum(q * q, axis=1, keepdims=True)
    p2 = jnp.sum(Pm * Pm, axis=1)[None, :]
    m = jnp.maximum((q2 + p2) + S, 1e-12)
    D = m * jax.lax.rsqrt(m)
    E = jnp.exp(-D)
    D_ref[...] = D
    E_ref[...] = E
    col = jax.lax.broadcasted_iota(jnp.int32, (M_, NP_), 0)
    grp = jax.lax.broadcasted_iota(jnp.int32, (M_, NP_), 1)
    G = jnp.where(col // NK_ == grp, 1.0, 0.0).astype(jnp.float32)
    ps = jax.lax.dot_general(
        E, G, (((1,), (0,)), ((), ())),
        preferred_element_type=jnp.float32)
    pred_ref[...] = ps / jnp.sum(ps, axis=1, keepdims=True)


@functools.partial(jax.jit, static_argnames=())
def kernel(q, P):
    B = q.shape[0]
    nb = B // BR_
    pred, D, E = pl.pallas_call(
        _body,
        grid=(nb,),
        in_specs=[
            pl.BlockSpec((BR_, EMBED_), lambda i: (i, 0)),
            pl.BlockSpec((M_, EMBED_), lambda i: (0, 0)),
        ],
        out_specs=[
            pl.BlockSpec((BR_, NP_), lambda i: (i, 0)),
            pl.BlockSpec((BR_, M_), lambda i: (i, 0)),
            pl.BlockSpec((BR_, M_), lambda i: (i, 0)),
        ],
        out_shape=[
            jax.ShapeDtypeStruct((B, NP_), jnp.float32),
            jax.ShapeDtypeStruct((B, M_), jnp.float32),
            jax.ShapeDtypeStruct((B, M_), jnp.float32),
        ],
        compiler_params=pltpu.CompilerParams(
            dimension_semantics=("parallel",)),
    )(q, P)
    return (pred, D, E.reshape(B, NP_, NK_))
